# baseline (speedup 1.0000x reference)
import functools

import jax
import jax.numpy as jnp
from jax import lax
from jax.experimental import pallas as pl
from jax.experimental.pallas import tpu as pltpu
from jax.experimental.pallas import tpu_sc as plsc

TOTAL_TOK = 16384
EMB = 128
_NSPLIT = 2
_TOK_SPLIT = TOTAL_TOK // _NSPLIT

_info = plsc.get_sparse_core_info()
_NC, _NS = _info.num_cores, _info.num_subcores
_NW = _NC * _NS
_B_PER_W = _TOK_SPLIT // _NW


def _gather_body(token_hbm, table_hbm, out_hbm, idx_v, rows_v):
    wid = lax.axis_index("s") * _NC + lax.axis_index("c")
    base = wid * _B_PER_W
    pltpu.sync_copy(token_hbm.at[pl.ds(base, _B_PER_W)], idx_v)
    pltpu.sync_copy(table_hbm.at[idx_v], rows_v)
    pltpu.sync_copy(rows_v, out_hbm.at[pl.ds(base, _B_PER_W)])


_mesh = plsc.VectorSubcoreMesh(core_axis_name="c", subcore_axis_name="s")

_gather = functools.partial(
    pl.kernel,
    mesh=_mesh,
    out_type=jax.ShapeDtypeStruct((_TOK_SPLIT, EMB), jnp.float32),
    scratch_types=[
        pltpu.VMEM((_B_PER_W,), jnp.int32),
        pltpu.VMEM((_B_PER_W, EMB), jnp.float32),
    ],
)(_gather_body)


@jax.jit
def kernel(token_ids, cu_seqlens, table):
    ids = token_ids.astype(jnp.int32)
    halves = [
        _gather(lax.slice(ids, (k * _TOK_SPLIT,), ((k + 1) * _TOK_SPLIT,)),
                table)
        for k in range(_NSPLIT)
    ]
    all_embs = jnp.concatenate(halves, axis=0)
    return (all_embs, cu_seqlens)


# speedup vs baseline: 1.4170x; 1.4170x over previous
import functools

import jax
import jax.numpy as jnp
from jax import lax
from jax.experimental import pallas as pl
from jax.experimental.pallas import tpu as pltpu
from jax.experimental.pallas import tpu_sc as plsc

TOTAL_TOK = 16384
EMB = 128

_info = plsc.get_sparse_core_info()
_NC, _NS = _info.num_cores, _info.num_subcores
_NW = _NC * _NS
_B_PER_W = TOTAL_TOK // _NW


def _gather_body(token_hbm, table_hbm, out_hbm, idx_v, rows_v):
    wid = lax.axis_index("s") * _NC + lax.axis_index("c")
    base = wid * _B_PER_W
    pltpu.sync_copy(token_hbm.at[pl.ds(base, _B_PER_W)], idx_v)
    pltpu.sync_copy(table_hbm.at[idx_v], rows_v)
    pltpu.sync_copy(rows_v, out_hbm.at[pl.ds(base, _B_PER_W)])


_mesh = plsc.VectorSubcoreMesh(core_axis_name="c", subcore_axis_name="s")

_gather = functools.partial(
    pl.kernel,
    mesh=_mesh,
    out_type=jax.ShapeDtypeStruct((TOTAL_TOK, EMB), jnp.float32),
    scratch_types=[
        pltpu.VMEM((_B_PER_W,), jnp.int32),
        pltpu.VMEM((_B_PER_W, EMB), jnp.float32),
    ],
)(_gather_body)


@jax.jit
def kernel(token_ids, cu_seqlens, table):
    all_embs = _gather(token_ids.astype(jnp.int32), table)
    return (all_embs, cu_seqlens)


# speedup vs baseline: 1.4198x; 1.0019x over previous
import functools

import jax
import jax.numpy as jnp
from jax import lax
from jax.experimental import pallas as pl
from jax.experimental.pallas import tpu as pltpu
from jax.experimental.pallas import tpu_sc as plsc

TOTAL_TOK = 16384
EMB = 128
NBOUND = 17

_info = plsc.get_sparse_core_info()
_NC, _NS = _info.num_cores, _info.num_subcores
_NW = _NC * _NS
_B_PER_W = TOTAL_TOK // _NW


def _gather_body(token_hbm, cu_hbm, table_hbm, out_hbm, out_cu_hbm,
                 idx_v, rows_v, cu_v):
    wid = lax.axis_index("s") * _NC + lax.axis_index("c")
    base = wid * _B_PER_W
    pltpu.sync_copy(token_hbm.at[pl.ds(base, _B_PER_W)], idx_v)
    pltpu.sync_copy(table_hbm.at[idx_v], rows_v)
    pltpu.sync_copy(rows_v, out_hbm.at[pl.ds(base, _B_PER_W)])

    @pl.when(wid == 0)
    def _():
        pltpu.sync_copy(cu_hbm, cu_v)
        pltpu.sync_copy(cu_v, out_cu_hbm)


_mesh = plsc.VectorSubcoreMesh(core_axis_name="c", subcore_axis_name="s")

_gather = functools.partial(
    pl.kernel,
    mesh=_mesh,
    out_type=(
        jax.ShapeDtypeStruct((TOTAL_TOK, EMB), jnp.float32),
        jax.ShapeDtypeStruct((NBOUND,), jnp.int32),
    ),
    scratch_types=[
        pltpu.VMEM((_B_PER_W,), jnp.int32),
        pltpu.VMEM((_B_PER_W, EMB), jnp.float32),
        pltpu.VMEM((NBOUND,), jnp.int32),
    ],
)(_gather_body)


@jax.jit
def kernel(token_ids, cu_seqlens, table):
    all_embs, boundaries = _gather(
        token_ids.astype(jnp.int32), cu_seqlens.astype(jnp.int32), table)
    return (all_embs, boundaries)
